# baseline (speedup 1.0000x reference)
import jax
import jax.numpy as jnp
from jax import lax
from jax.experimental import pallas as pl
from jax.experimental.pallas import tpu as pltpu
from jax.experimental.pallas import tpu_sc as plsc

B, L = 4096, 200
N = B * L
D0, D1, D2 = 64, 32, 32
DO = D0 + D1 + D2
W = 32
R0, R1, R2 = 4096, 2048, 512
NC, NS = 2, 16
NW = NC * NS
C = 400
K = 4 * C
TOK_PER_W = N // NW
CHUNKS_PER_W = TOK_PER_W // C
LANES = 16


def _encoder_kernel(ids0, ids1, ids2, t0, t1, t2, out,
                    tc_s,
                    ids0_v, ids1_v, ids2_v,
                    idx_v, out_v,
                    sem_i, sem_g, sem_w):
    cid = lax.axis_index("c")
    sid = lax.axis_index("s")
    wid = sid * NC + cid

    @pl.when(sid == 0)
    def _stage():
        pltpu.sync_copy(t0, tc_s.at[pl.ds(0, R0)])
        pltpu.sync_copy(t1, tc_s.at[pl.ds(R0, R1)])
        pltpu.sync_copy(t2, tc_s.at[pl.ds(R0 + R1, R2)])

    plsc.subcore_barrier()

    row0 = wid * CHUNKS_PER_W

    def fire_idx_fetch(c, p):
        r = row0 + lax.rem(c, CHUNKS_PER_W)
        pltpu.async_copy(ids0.at[r], ids0_v.at[p], sem_i.at[p])
        pltpu.async_copy(ids1.at[r], ids1_v.at[p], sem_i.at[p])
        pltpu.async_copy(ids2.at[r], ids2_v.at[p], sem_i.at[p])

    def wait_idx_fetch(p):
        pltpu.make_async_copy(ids0.at[0], ids0_v.at[p], sem_i.at[p]).wait()
        pltpu.make_async_copy(ids1.at[0], ids1_v.at[p], sem_i.at[p]).wait()
        pltpu.make_async_copy(ids2.at[0], ids2_v.at[p], sem_i.at[p]).wait()

    def compute_idx(p):
        lane = lax.iota(jnp.int32, LANES)

        def blk(k, carry):
            t16 = pl.ds(k * LANES, LANES)
            pos = 4 * (k * LANES + lane)
            v0 = ids0_v[p, t16]
            v1 = ids1_v[p, t16]
            v2 = ids2_v[p, t16]
            tgt = idx_v.at[p]
            plsc.store_scatter(tgt, [pos], 2 * v0)
            plsc.store_scatter(tgt, [pos + 1], 2 * v0 + 1)
            plsc.store_scatter(tgt, [pos + 2], v1 + R0)
            plsc.store_scatter(tgt, [pos + 3], v2 + (R0 + R1))
            return carry

        lax.fori_loop(0, C // LANES, blk, 0)

    def wait_gather(p):
        pltpu.make_async_copy(tc_s.at[idx_v.at[p]], out_v.at[p],
                              sem_g.at[p]).wait()

    def fire_write(c, p):
        base = (row0 + c) * K
        pltpu.async_copy(out_v.at[p], out.at[pl.ds(base, K)], sem_w.at[p])

    def wait_write(p):
        pltpu.make_async_copy(out_v.at[p], out.at[pl.ds(0, K)],
                              sem_w.at[p]).wait()

    def chunk_step(c, p):
        q = 1 - p
        @pl.when(c >= 2)
        def _():
            wait_write(p)
        pltpu.async_copy(tc_s.at[idx_v.at[p]], out_v.at[p], sem_g.at[p])

        @pl.when(c >= 1)
        def _():
            wait_gather(q)
            fire_write(c - 1, q)

        wait_idx_fetch(q)
        compute_idx(q)
        fire_idx_fetch(c + 2, p)

    fire_idx_fetch(0, 0)
    wait_idx_fetch(0)
    compute_idx(0)
    fire_idx_fetch(1, 1)

    def body(i, carry):
        chunk_step(2 * i, 0)
        chunk_step(2 * i + 1, 1)
        return carry

    lax.fori_loop(0, CHUNKS_PER_W // 2, body, 0)

    wait_gather(1)
    fire_write(CHUNKS_PER_W - 1, 1)
    wait_idx_fetch(1)
    wait_write(0)
    wait_write(1)


def kernel(pos_ids_0, pos_ids_1, pos_ids_2, table_0, table_1, table_2):
    ids0 = pos_ids_0.reshape(N // C, C)
    ids1 = pos_ids_1.reshape(N // C, C)
    ids2 = pos_ids_2.reshape(N // C, C)
    t0 = table_0.reshape(R0, W)

    mesh = plsc.VectorSubcoreMesh(core_axis_name="c", subcore_axis_name="s")
    run = pl.kernel(
        _encoder_kernel,
        out_type=jax.ShapeDtypeStruct((4 * N, W), jnp.float32),
        mesh=mesh,
        compiler_params=pltpu.CompilerParams(use_tc_tiling_on_sc=False,
                                             needs_layout_passes=False),
        scratch_types=[
            pltpu.VMEM_SHARED((R0 + R1 + R2, W), jnp.float32),
            pltpu.VMEM((2, C), jnp.int32),
            pltpu.VMEM((2, C), jnp.int32),
            pltpu.VMEM((2, C), jnp.int32),
            pltpu.VMEM((2, K), jnp.int32),
            pltpu.VMEM((2, K, W), jnp.float32),
            pltpu.SemaphoreType.DMA((2,)),
            pltpu.SemaphoreType.DMA((2,)),
            pltpu.SemaphoreType.DMA((2,)),
        ],
    )
    out = run(ids0, ids1, ids2, t0, table_1, table_2)
    return out.reshape(B, L, DO)


# speedup vs baseline: 1.0078x; 1.0078x over previous
import jax
import jax.numpy as jnp
from jax import lax
from jax.experimental import pallas as pl
from jax.experimental.pallas import tpu as pltpu
from jax.experimental.pallas import tpu_sc as plsc

B, L = 4096, 200
N = B * L
D0, D1, D2 = 64, 32, 32
DO = D0 + D1 + D2
NC, NS = 2, 16
NW = NC * NS
C = 400
TOK_PER_W = N // NW
CHUNKS_PER_W = TOK_PER_W // C


def _encoder_kernel(ids0, ids1, ids2, t0, t1, t2, out,
                    t0_s, t1_s, t2_s,
                    idx0_v, idx1_v, idx2_v,
                    e0_v, e1_v, e2_v,
                    sem_i, sem_g, sem_w):
    cid = lax.axis_index("c")
    sid = lax.axis_index("s")
    wid = sid * NC + cid

    @pl.when(sid == 0)
    def _stage():
        pltpu.sync_copy(t0, t0_s)
        pltpu.sync_copy(t1, t1_s)
        pltpu.sync_copy(t2, t2_s)

    plsc.subcore_barrier()

    row0 = wid * CHUNKS_PER_W

    def fire_idx_fetch(c, p):
        r = row0 + lax.rem(c, CHUNKS_PER_W)
        pltpu.async_copy(ids0.at[r], idx0_v.at[p], sem_i.at[p])
        pltpu.async_copy(ids1.at[r], idx1_v.at[p], sem_i.at[p])
        pltpu.async_copy(ids2.at[r], idx2_v.at[p], sem_i.at[p])

    def wait_idx_fetch(p):
        pltpu.make_async_copy(ids0.at[0], idx0_v.at[p], sem_i.at[p]).wait()
        pltpu.make_async_copy(ids1.at[0], idx1_v.at[p], sem_i.at[p]).wait()
        pltpu.make_async_copy(ids2.at[0], idx2_v.at[p], sem_i.at[p]).wait()

    def out_slices(base):
        return (out.at[pl.ds(base, C), pl.ds(0, D0)],
                out.at[pl.ds(base, C), pl.ds(D0, D1)],
                out.at[pl.ds(base, C), pl.ds(D0 + D1, D2)])

    def wait_writes(p, e0, e1, e2):
        o0, o1, o2 = out_slices(0)
        pltpu.make_async_copy(e0, o0, sem_w.at[p]).wait()
        pltpu.make_async_copy(e1, o1, sem_w.at[p]).wait()
        pltpu.make_async_copy(e2, o2, sem_w.at[p]).wait()

    def wait_gathers(p, e0, e1, e2):
        pltpu.make_async_copy(t0_s.at[idx0_v.at[p]], e0, sem_g.at[p]).wait()
        pltpu.make_async_copy(t1_s.at[idx1_v.at[p]], e1, sem_g.at[p]).wait()
        pltpu.make_async_copy(t2_s.at[idx2_v.at[p]], e2, sem_g.at[p]).wait()

    def chunk_step(c, p):
        q = 1 - p
        e0, e1, e2 = e0_v.at[p], e1_v.at[p], e2_v.at[p]
        f0, f1, f2 = e0_v.at[q], e1_v.at[q], e2_v.at[q]
        wait_idx_fetch(p)
        @pl.when(c >= 2)
        def _():
            wait_writes(p, e0, e1, e2)
        pltpu.async_copy(t0_s.at[idx0_v.at[p]], e0, sem_g.at[p])
        pltpu.async_copy(t1_s.at[idx1_v.at[p]], e1, sem_g.at[p])
        pltpu.async_copy(t2_s.at[idx2_v.at[p]], e2, sem_g.at[p])

        @pl.when(c >= 1)
        def _():
            wait_gathers(q, f0, f1, f2)
            fire_idx_fetch(c + 1, q)
            base = (row0 + c - 1) * C
            o0, o1, o2 = out_slices(base)
            pltpu.async_copy(f0, o0, sem_w.at[q])
            pltpu.async_copy(f1, o1, sem_w.at[q])
            pltpu.async_copy(f2, o2, sem_w.at[q])

        @pl.when(c == 0)
        def _():
            fire_idx_fetch(1, q)

    fire_idx_fetch(0, 0)

    def body(i, carry):
        chunk_step(2 * i, 0)
        chunk_step(2 * i + 1, 1)
        return carry

    lax.fori_loop(0, CHUNKS_PER_W // 2, body, 0)

    lastp = 1
    el0, el1, el2 = e0_v.at[lastp], e1_v.at[lastp], e2_v.at[lastp]
    wait_gathers(lastp, el0, el1, el2)
    base = (row0 + CHUNKS_PER_W - 1) * C
    o0, o1, o2 = out_slices(base)
    pltpu.async_copy(el0, o0, sem_w.at[lastp])
    pltpu.async_copy(el1, o1, sem_w.at[lastp])
    pltpu.async_copy(el2, o2, sem_w.at[lastp])
    wait_writes(0, e0_v.at[0], e1_v.at[0], e2_v.at[0])
    wait_writes(1, el0, el1, el2)
    wait_idx_fetch(0)


def kernel(pos_ids_0, pos_ids_1, pos_ids_2, table_0, table_1, table_2):
    ids0 = pos_ids_0.reshape(N // C, C)
    ids1 = pos_ids_1.reshape(N // C, C)
    ids2 = pos_ids_2.reshape(N // C, C)

    mesh = plsc.VectorSubcoreMesh(core_axis_name="c", subcore_axis_name="s")
    run = pl.kernel(
        _encoder_kernel,
        out_type=jax.ShapeDtypeStruct((N, DO), jnp.float32),
        mesh=mesh,
        compiler_params=pltpu.CompilerParams(use_tc_tiling_on_sc=False,
                                             skip_device_barrier=True),
        scratch_types=[
            pltpu.VMEM_SHARED((2048, D0), jnp.float32),
            pltpu.VMEM_SHARED((2048, D1), jnp.float32),
            pltpu.VMEM_SHARED((512, D2), jnp.float32),
            pltpu.VMEM((2, C), jnp.int32),
            pltpu.VMEM((2, C), jnp.int32),
            pltpu.VMEM((2, C), jnp.int32),
            pltpu.VMEM((2, C, D0), jnp.float32),
            pltpu.VMEM((2, C, D1), jnp.float32),
            pltpu.VMEM((2, C, D2), jnp.float32),
            pltpu.SemaphoreType.DMA((2,)),
            pltpu.SemaphoreType.DMA((2,)),
            pltpu.SemaphoreType.DMA((2,)),
        ],
    )
    out = run(ids0, ids1, ids2, table_0, table_1, table_2)
    return out.reshape(B, L, DO)
